# Initial kernel scaffold; baseline (speedup 1.0000x reference)
import jax
import jax.numpy as jnp
from jax.experimental import pallas as pl


def kernel(logits, value):
    raise NotImplementedError("write your pallas kernel here")


# speedup vs baseline: 1.2383x; 1.2383x over previous
import functools

import jax
import jax.numpy as jnp
from jax import lax
from jax.experimental import pallas as pl
from jax.experimental.pallas import tpu as pltpu
from jax.experimental.pallas import tpu_sc as plsc

_A = 26
_N = 100000
_B = 4096
_NC = 2
_NS = 16
_NW = _NC * _NS
_BW = _B // _NW
_L = 16


def _lse_body(x_ref, out_ref):
    x = x_ref[...]
    m = jnp.max(x, axis=1, keepdims=True)
    s = jnp.sum(jnp.exp(x - m), axis=1, keepdims=True)
    out_ref[0, 0] = jnp.sum(m + jnp.log(s))


def _lse_sum(logits):
    return pl.pallas_call(
        _lse_body,
        out_shape=jax.ShapeDtypeStruct((1, 1), jnp.float32),
        out_specs=pl.BlockSpec(memory_space=pltpu.SMEM),
    )(logits)


@functools.lru_cache(maxsize=1)
def _make_gather_kernel():
    mesh = plsc.VectorSubcoreMesh(core_axis_name="c", subcore_axis_name="s")

    @functools.partial(
        pl.kernel,
        mesh=mesh,
        out_type=jax.ShapeDtypeStruct((_B,), jnp.float32),
        scratch_types=[
            pltpu.VMEM((_A, _BW), jnp.int32),
            pltpu.VMEM((_A, _BW), jnp.int32),
            pltpu.VMEM((_A, _BW), jnp.float32),
            pltpu.VMEM((_BW,), jnp.float32),
            pltpu.SemaphoreType.DMA,
        ],
    )
    def k(vt_hbm, logits_hbm, out_hbm, vv, iv, gv, ov, sem):
        wid = lax.axis_index("s") * _NC + lax.axis_index("c")
        pltpu.sync_copy(vt_hbm.at[wid], vv)
        for a in range(_A):
            for j in range(_BW // _L):
                sl = pl.ds(j * _L, _L)
                iv[a, sl] = vv[a, sl] + a * _N
        descs = []
        for a in range(_A):
            descs.append(pltpu.async_copy(logits_hbm.at[iv.at[a]], gv.at[a], sem))
        for d in descs:
            d.wait()
        for j in range(_BW // _L):
            sl = pl.ds(j * _L, _L)
            acc = gv[0, sl]
            for a in range(1, _A):
                acc = acc + gv[a, sl]
            ov[sl] = acc
        pltpu.sync_copy(ov, out_hbm.at[pl.ds(wid * _BW, _BW)])

    return k


def kernel(logits, value):
    vt = value.T.reshape(_A, _NW, _BW).transpose(1, 0, 2)
    gsum = _make_gather_kernel()(vt, logits.reshape(-1))
    c = _lse_sum(logits)[0, 0]
    return gsum - c


# speedup vs baseline: 1.6880x; 1.3631x over previous
import functools

import jax
import jax.numpy as jnp
from jax import lax
from jax.experimental import pallas as pl
from jax.experimental.pallas import tpu as pltpu
from jax.experimental.pallas import tpu_sc as plsc

_A = 26
_N = 100000
_B = 4096
_NC = 2
_NS = 16
_L = 16
_ROWS = _B // 128


def _lse_body(x_ref, out_ref):
    x = x_ref[...]
    m = jnp.max(x, axis=1, keepdims=True)
    s = jnp.sum(jnp.exp(x - m), axis=1, keepdims=True)
    out_ref[0, 0] = jnp.sum(m + jnp.log(s))


def _lse_sum(logits):
    return pl.pallas_call(
        _lse_body,
        out_shape=jax.ShapeDtypeStruct((1, 1), jnp.float32),
        out_specs=pl.BlockSpec(memory_space=pltpu.SMEM),
    )(logits)


@functools.lru_cache(maxsize=1)
def _make_gather_kernel():
    mesh = plsc.VectorSubcoreMesh(core_axis_name="c", subcore_axis_name="s")

    @functools.partial(
        pl.kernel,
        mesh=mesh,
        compiler_params=pltpu.CompilerParams(needs_layout_passes=False),
        out_type=jax.ShapeDtypeStruct((_NC, _ROWS, 128), jnp.float32),
        scratch_types=[
            pltpu.VMEM((_N,), jnp.float32),
            pltpu.VMEM((_B,), jnp.int32),
            pltpu.VMEM((_ROWS, 128), jnp.float32),
            pltpu.VMEM((_ROWS,), jnp.int32),
            pltpu.VMEM_SHARED((_ROWS, 128), jnp.float32),
        ],
    )
    def k(logits_hbm, vt_hbm, out_hbm, row_v, idx_v, part_v, sidx_v, shared):
        cid = lax.axis_index("c")
        sid = lax.axis_index("s")
        row = cid * _NS + sid
        active = row < _A

        sidx_v[pl.ds(0, _L)] = lax.iota(jnp.int32, _L)
        sidx_v[pl.ds(_L, _L)] = lax.iota(jnp.int32, _L) + _L

        @pl.when(active)
        def _():
            pltpu.sync_copy(vt_hbm.at[row], idx_v)
            pltpu.sync_copy(logits_hbm.at[row], row_v)
            for i in range(_B // _L):
                g = plsc.load_gather(row_v, [idx_v[pl.ds(i * _L, _L)]])
                part_v[i // 8, pl.ds((i % 8) * _L, _L)] = g

        @pl.when(sid == 0)
        def _():
            pltpu.sync_copy(part_v, shared)

        plsc.subcore_barrier()

        @pl.when(jnp.logical_and(active, sid != 0))
        def _():
            pltpu.sync_copy(part_v, shared.at[sidx_v], add=True)

        plsc.subcore_barrier()

        @pl.when(sid == 0)
        def _():
            pltpu.sync_copy(shared, out_hbm.at[cid])

    return k


def kernel(logits, value):
    partials = _make_gather_kernel()(logits, value.T)
    c = _lse_sum(logits)[0, 0]
    p = partials.reshape(_NC, _B)
    return p[0] + p[1] - c


# speedup vs baseline: 1.7369x; 1.0290x over previous
import functools

import jax
import jax.numpy as jnp
from jax import lax
from jax.experimental import pallas as pl
from jax.experimental.pallas import tpu as pltpu
from jax.experimental.pallas import tpu_sc as plsc

_A = 26
_N = 100000
_B = 4096
_NC = 2
_NS = 16
_L = 16
_ROWS = _B // 128


def _lse_body(x_ref, out_ref):
    x = x_ref[...]
    m = jnp.max(x, axis=1, keepdims=True)
    s = jnp.sum(jnp.exp(x - m), axis=1, keepdims=True)
    out_ref[0, 0] = jnp.sum(m + jnp.log(s))


def _lse_sum(logits):
    return pl.pallas_call(
        _lse_body,
        out_shape=jax.ShapeDtypeStruct((1, 1), jnp.float32),
        out_specs=pl.BlockSpec(memory_space=pltpu.SMEM),
    )(logits)


@functools.lru_cache(maxsize=1)
def _make_gather_kernel():
    mesh = plsc.VectorSubcoreMesh(core_axis_name="c", subcore_axis_name="s")

    @functools.partial(
        pl.kernel,
        mesh=mesh,
        compiler_params=pltpu.CompilerParams(needs_layout_passes=False),
        out_type=[
            jax.ShapeDtypeStruct((_ROWS, 128), jnp.float32),
            jax.ShapeDtypeStruct((_ROWS, 128), jnp.float32),
        ],
        scratch_types=[
            pltpu.VMEM((_N,), jnp.float32),
            pltpu.VMEM((_B,), jnp.int32),
            pltpu.VMEM((_ROWS, 128), jnp.float32),
            pltpu.VMEM((_ROWS,), jnp.int32),
            pltpu.VMEM_SHARED((_ROWS, 128), jnp.float32),
        ],
    )
    def k(logits_hbm, vt_hbm, out_a, out_b, row_v, idx_v, part_v, sidx_v, shared):
        cid = lax.axis_index("c")
        sid = lax.axis_index("s")
        row = cid * 13 + sid
        active = sid < 13

        sidx_v[pl.ds(0, _L)] = lax.iota(jnp.int32, _L)
        sidx_v[pl.ds(_L, _L)] = lax.iota(jnp.int32, _L) + _L

        @pl.when(active)
        def _():
            pltpu.sync_copy(vt_hbm.at[row], idx_v)
            pltpu.sync_copy(logits_hbm.at[row], row_v)
            for i in range(_B // _L):
                g = plsc.load_gather(row_v, [idx_v[pl.ds(i * _L, _L)]])
                part_v[i // 8, pl.ds((i % 8) * _L, _L)] = g

        @pl.when(sid == 0)
        def _():
            pltpu.sync_copy(part_v, shared)

        plsc.subcore_barrier()

        @pl.when(jnp.logical_and(active, sid != 0))
        def _():
            pltpu.sync_copy(part_v, shared.at[sidx_v], add=True)

        plsc.subcore_barrier()

        @pl.when(jnp.logical_and(sid == 0, cid == 0))
        def _():
            pltpu.sync_copy(shared, out_a)

        @pl.when(jnp.logical_and(sid == 0, cid == 1))
        def _():
            pltpu.sync_copy(shared, out_b)

    return k


def kernel(logits, value):
    pa, pb = _make_gather_kernel()(logits, value.T)
    c = _lse_sum(logits)[0, 0]
    return (pa + pb - c).reshape(_B)


# speedup vs baseline: 1.9013x; 1.0947x over previous
import functools

import jax
import jax.numpy as jnp
from jax import lax
from jax.experimental import pallas as pl
from jax.experimental.pallas import tpu as pltpu
from jax.experimental.pallas import tpu_sc as plsc

_A = 26
_N = 100000
_B = 4096
_NC = 2
_NS = 16
_L = 16
_ROWS = _B // 128


def _lse_body(x_ref, out_ref):
    x = x_ref[...]
    m = jnp.max(x, axis=1, keepdims=True)
    s = jnp.sum(jnp.exp(x - m), axis=1, keepdims=True)
    out_ref[0, 0] = jnp.sum(m + jnp.log(s))


def _lse_sum(logits):
    return pl.pallas_call(
        _lse_body,
        out_shape=jax.ShapeDtypeStruct((1, 1), jnp.float32),
        out_specs=pl.BlockSpec(memory_space=pltpu.SMEM),
    )(logits)


@functools.lru_cache(maxsize=1)
def _make_gather_kernel():
    mesh = plsc.VectorSubcoreMesh(core_axis_name="c", subcore_axis_name="s")

    @functools.partial(
        pl.kernel,
        mesh=mesh,
        compiler_params=pltpu.CompilerParams(needs_layout_passes=False),
        out_type=[
            jax.ShapeDtypeStruct((_ROWS, 128), jnp.float32),
            jax.ShapeDtypeStruct((_ROWS, 128), jnp.float32),
        ],
        scratch_types=[
            pltpu.VMEM((_N,), jnp.float32),
            pltpu.VMEM((_B,), jnp.int32),
            pltpu.VMEM((_ROWS, 128), jnp.float32),
            pltpu.VMEM((_ROWS,), jnp.int32),
            pltpu.VMEM_SHARED((_ROWS, 128), jnp.float32),
            pltpu.SemaphoreType.DMA,
        ],
    )
    def k(logits_hbm, vt_hbm, out_a, out_b, row_v, idx_v, part_v, sidx_v, shared,
          sem):
        cid = lax.axis_index("c")
        sid = lax.axis_index("s")
        row = cid * 13 + sid
        active = sid < 13

        sidx_v[pl.ds(0, _L)] = lax.iota(jnp.int32, _L)
        sidx_v[pl.ds(_L, _L)] = lax.iota(jnp.int32, _L) + _L

        @pl.when(active)
        def _():
            cp = pltpu.async_copy(logits_hbm.at[row], row_v, sem)
            pltpu.sync_copy(vt_hbm.at[row], idx_v)
            cp.wait()

            @plsc.parallel_loop(0, _ROWS, step=1, unroll=2)
            def _(r):
                for j in range(8):
                    idx16 = idx_v[pl.ds(r * 128 + j * _L, _L)]
                    g = plsc.load_gather(row_v, [idx16])
                    part_v[r, pl.ds(j * _L, _L)] = g

        @pl.when(sid == 0)
        def _():
            pltpu.sync_copy(part_v, shared)

        plsc.subcore_barrier()

        @pl.when(jnp.logical_and(active, sid != 0))
        def _():
            pltpu.sync_copy(part_v, shared.at[sidx_v], add=True)

        plsc.subcore_barrier()

        @pl.when(jnp.logical_and(sid == 0, cid == 0))
        def _():
            pltpu.sync_copy(shared, out_a)

        @pl.when(jnp.logical_and(sid == 0, cid == 1))
        def _():
            pltpu.sync_copy(shared, out_b)

    return k


def kernel(logits, value):
    pa, pb = _make_gather_kernel()(logits, value.T)
    c = _lse_sum(logits)[0, 0]
    return (pa + pb - c).reshape(_B)
